# Initial kernel scaffold; baseline (speedup 1.0000x reference)
import jax
import jax.numpy as jnp
from jax.experimental import pallas as pl


def kernel(x, W, bias, d_vals, adj_vals, diag_vals, edge_index):
    raise NotImplementedError("write your pallas kernel here")


# speedup vs baseline: 2.8204x; 2.8204x over previous
import functools

import jax
import jax.numpy as jnp
from jax import lax
from jax.experimental import pallas as pl
from jax.experimental.pallas import tpu as pltpu
from jax.experimental.pallas import tpu_sc as plsc

N = 10000
NPAD = 10240
F = 128
R = 3
NC = 2
NS = 16
K = 128
NB = 82
EW = K * NB
EPAD = NC * NS * EW
ROWS_PER_TILE = NPAD // NS
FCHUNK = 128
NCHUNK = ROWS_PER_TILE // FCHUNK


def _make_sc_spmm(num_tables, vals_per_round, table_per_round, fuse_rounds):
    mesh = plsc.VectorSubcoreMesh(core_axis_name="c", subcore_axis_name="s")
    if fuse_rounds:
        out_type = jax.ShapeDtypeStruct((NC, NPAD, F), jnp.float32)
    else:
        out_type = jax.ShapeDtypeStruct((num_tables, NC, NPAD, F), jnp.float32)

    @functools.partial(
        pl.kernel,
        out_type=out_type,
        mesh=mesh,
        scratch_types=[
            pltpu.VMEM((K,), jnp.int32),
            pltpu.VMEM((K,), jnp.int32),
            pltpu.VMEM((K,), jnp.float32),
            pltpu.VMEM((K, F), jnp.float32),
            pltpu.VMEM((FCHUNK, F), jnp.float32),
            pltpu.VMEM_SHARED((NPAD, F), jnp.float32),
            pltpu.SemaphoreType.DMA,
        ],
    )
    def kern(tables, cols, rows, vals, zeros, out, colv, rowv, valv, gbuf,
             zbuf, acc, sem):
        cid = lax.axis_index("c")
        sid = lax.axis_index("s")
        wid = cid * NS + sid
        base = wid * EW
        tile_row0 = sid * ROWS_PER_TILE

        def zero_acc():
            pltpu.sync_copy(zeros, zbuf)
            for z in range(NCHUNK):
                pltpu.sync_copy(zbuf, acc.at[pl.ds(tile_row0 + z * FCHUNK, FCHUNK)])
            plsc.subcore_barrier()

        def edge_pass(ii):
            vi = ii if vals_per_round else 0
            ti = ii if table_per_round else 0

            def block(b, carry):
                off = base + b * K
                pltpu.sync_copy(cols.at[pl.ds(off, K)], colv)
                pltpu.sync_copy(rows.at[pl.ds(off, K)], rowv)
                pltpu.sync_copy(vals.at[pl.ds(vi * EPAD + off, K)], valv)
                cp = pltpu.async_copy(tables.at[ti].at[colv], gbuf, sem)
                cp.wait()

                def group(g, c2):
                    vgrp = valv[pl.ds(g * 16, 16)]
                    for j in range(16):
                        e = g * 16 + j
                        v = lax.gather(
                            vgrp, jnp.full((16, 1), j, jnp.int32),
                            lax.GatherDimensionNumbers(
                                offset_dims=(), collapsed_slice_dims=(0,),
                                start_index_map=(0,)),
                            slice_sizes=(1,),
                            mode=lax.GatherScatterMode.PROMISE_IN_BOUNDS)
                        for f in range(F // 16):
                            sl = pl.ds(f * 16, 16)
                            gbuf[e, sl] = gbuf[e, sl] * v
                    return c2

                lax.fori_loop(0, K // 16, group, 0)
                pltpu.sync_copy(gbuf, acc.at[rowv], add=True)
                return carry

            lax.fori_loop(0, NB, block, 0)
            plsc.subcore_barrier()

        def flush(out_view):
            for z in range(NCHUNK):
                r0 = tile_row0 + z * FCHUNK
                pltpu.sync_copy(acc.at[pl.ds(r0, FCHUNK)], zbuf)
                pltpu.sync_copy(zbuf, out_view.at[pl.ds(r0, FCHUNK)])

        if fuse_rounds:
            zero_acc()
            for ii in range(num_tables):
                edge_pass(ii)
            flush(out.at[cid])
        else:
            for ii in range(num_tables):
                zero_acc()
                edge_pass(ii)
                flush(out.at[ii, cid])
                plsc.subcore_barrier()

    return kern


_sc_spmm_x = _make_sc_spmm(R, vals_per_round=True, table_per_round=False,
                           fuse_rounds=False)
_sc_spmm_adj = _make_sc_spmm(R, vals_per_round=False, table_per_round=True,
                             fuse_rounds=False)
_sc_spmm_fused = _make_sc_spmm(R, vals_per_round=True, table_per_round=True,
                               fuse_rounds=True)


BN = 1024
GRID_N = NPAD // BN


def _mm_body(p_ref, w_ref, o_ref):
    s = p_ref[0, 0] + p_ref[0, 1]
    o_ref[0] = jnp.dot(s, w_ref[0], preferred_element_type=jnp.float32)


def _tc_merge_matmul(parts, w):
    return pl.pallas_call(
        _mm_body,
        grid=(R, GRID_N),
        in_specs=[
            pl.BlockSpec((1, NC, BN, F), lambda r, b: (r, 0, b, 0)),
            pl.BlockSpec((1, F, F), lambda r, b: (r, 0, 0)),
        ],
        out_specs=pl.BlockSpec((1, BN, F), lambda r, b: (r, b, 0)),
        out_shape=jax.ShapeDtypeStruct((R, NPAD, F), jnp.float32),
    )(parts, w)


def _relu_body(u_ref, t1_ref, diag_ref, o_ref):
    r = pl.program_id(0)
    sign = jnp.where(r == 0, -1.0, 1.0)
    u = u_ref[0, 0] + u_ref[0, 1]
    o_ref[0] = jnp.maximum(u + sign * diag_ref[...] * t1_ref[0], 0.0)


def _tc_relu(u_parts, t1, diag2):
    return pl.pallas_call(
        _relu_body,
        grid=(R, GRID_N),
        in_specs=[
            pl.BlockSpec((1, NC, BN, F), lambda r, b: (r, 0, b, 0)),
            pl.BlockSpec((1, BN, F), lambda r, b: (r, b, 0)),
            pl.BlockSpec((BN, 1), lambda r, b: (b, 0)),
        ],
        out_specs=pl.BlockSpec((1, BN, F), lambda r, b: (r, b, 0)),
        out_shape=jax.ShapeDtypeStruct((R, NPAD, F), jnp.float32),
    )(u_parts, t1, diag2)


def _final_body(p_ref, b_ref, o_ref):
    o_ref[...] = p_ref[0] + p_ref[1] + b_ref[...]


def _tc_final(parts, bias):
    return pl.pallas_call(
        _final_body,
        grid=(GRID_N,),
        in_specs=[
            pl.BlockSpec((NC, BN, F), lambda b: (0, b, 0)),
            pl.BlockSpec((1, F), lambda b: (0, 0)),
        ],
        out_specs=pl.BlockSpec((BN, F), lambda b: (b, 0)),
        out_shape=jax.ShapeDtypeStruct((NPAD, F), jnp.float32),
    )(parts, bias)


def kernel(x, W, bias, d_vals, adj_vals, diag_vals, edge_index):
    rows = edge_index[0]
    cols = edge_index[1]
    E = rows.shape[0]
    pad = EPAD - E
    cols_p = jnp.pad(cols, (0, pad))
    rows_p = jnp.pad(rows, (0, pad))
    d_p = jnp.pad(d_vals, ((0, 0), (0, pad))).reshape(-1)
    adj_p = jnp.pad(adj_vals, (0, pad))
    x_p = jnp.pad(x, ((0, NPAD - N), (0, 0)))
    diag2 = jnp.pad(diag_vals, (0, NPAD - N))[:, None]
    zeros = jnp.zeros((FCHUNK, F), jnp.float32)

    s_parts = _sc_spmm_x(x_p[None], cols_p, rows_p, d_p, zeros)
    t1 = _tc_merge_matmul(s_parts, W)
    u_parts = _sc_spmm_adj(t1, cols_p, rows_p, adj_p, zeros)
    v = _tc_relu(u_parts, t1, diag2)
    out_parts = _sc_spmm_fused(v, cols_p, rows_p, d_p, zeros)
    out = _tc_final(out_parts, bias[None, :])
    return out[:N]
